# baseline (speedup 1.0000x reference)
import dataclasses
import functools

import jax
import jax.numpy as jnp
from jax import lax
from jax.experimental import pallas as pl
from jax.experimental.pallas import tpu as pltpu
from jax.experimental.pallas import tpu_sc as plsc

_B, _W, _H, _L = 4, 32, 32, 32
_N = _B * _W * _H * _L
_D = 256
_NUM_EMB = (16, 8, 4, 6)
_OFFS = (0, 16, 24, 28)
_TBL_ROWS = 40
_SENT_ROW = 34

_COUNT = 13107
_VPAD = 13312
_SENT_SLOT = 13200
_VTOT = 4 * _VPAD

_ROWS, _COLS = 512, 256

_NC, _NS, _LANES = 2, 16, 16
_NW = _NC * _NS
_HROWS = _ROWS // 2
_NH = _N // 2
_PER_TILE = (4 * _NH) // _NW


def _rank_body(m_ref, rank_ref_a, rank_ref_b):
    iota_r = lax.broadcasted_iota(jnp.int32, (_COLS, _COLS), 0)
    iota_c = lax.broadcasted_iota(jnp.int32, (_COLS, _COLS), 1)
    upper = (iota_r <= iota_c).astype(jnp.float32)
    iota_r2 = lax.broadcasted_iota(jnp.int32, (_ROWS, _ROWS), 0)
    iota_c2 = lax.broadcasted_iota(jnp.int32, (_ROWS, _ROWS), 1)
    strict_lower = (iota_c2 < iota_r2).astype(jnp.float32)
    for i in range(4):
        m = m_ref[i]
        m_f = m.astype(jnp.float32)
        row_incl = lax.dot(m_f, upper, precision=lax.Precision.HIGHEST)
        row_tot = row_incl[:, _COLS - 1:_COLS]
        col_excl = lax.dot(strict_lower, row_tot,
                           precision=lax.Precision.HIGHEST)
        excl = row_incl - m_f + col_excl
        rank = excl.astype(jnp.int32)
        rank = jnp.where(m == 1, rank, _SENT_SLOT)
        rank_ref_a[i] = rank[:_HROWS]
        rank_ref_b[i] = rank[_HROWS:]


def _ranks(masks_i32):
    half = jax.ShapeDtypeStruct((4, _HROWS, _COLS), jnp.int32)
    return pl.pallas_call(
        _rank_body,
        out_shape=(half, half),
    )(masks_i32)


def _sc_compiler_params():
    cp = pltpu.CompilerParams()
    if "needs_layout_passes" in pltpu.CompilerParams.__dataclass_fields__:
        cp = dataclasses.replace(cp, needs_layout_passes=False)
    return cp


def _gather_body(vals_hbm, idx_hbm, out_hbm, vals_v, idx_v, out_v, sem, sem2):
    wid = lax.axis_index("s") * _NC + lax.axis_index("c")
    head = wid // (_NW // 4)
    base = wid * _PER_TILE
    c1 = pltpu.async_copy(vals_hbm.at[pl.ds(head * _VPAD, _VPAD)], vals_v, sem)
    c2 = pltpu.async_copy(idx_hbm.at[pl.ds(base, _PER_TILE)], idx_v, sem2)
    c1.wait()
    c2.wait()

    @plsc.parallel_loop(0, _PER_TILE, _LANES, unroll=8)
    def _(i):
        idxv = idx_v[pl.ds(i, _LANES)]
        out_v[pl.ds(i, _LANES)] = plsc.load_gather(vals_v, [idxv])

    pltpu.async_copy(out_v, out_hbm.at[pl.ds(base, _PER_TILE)], sem).wait()


def _sc_gather(vals_all, rank_flat):
    mesh = plsc.VectorSubcoreMesh(core_axis_name="c", subcore_axis_name="s")
    k = pl.kernel(
        _gather_body,
        out_type=jax.ShapeDtypeStruct((4 * _NH,), jnp.int32),
        mesh=mesh,
        scratch_types=[
            pltpu.VMEM((_VPAD,), jnp.int32),
            pltpu.VMEM((_PER_TILE,), jnp.int32),
            pltpu.VMEM((_PER_TILE,), jnp.int32),
            pltpu.SemaphoreType.DMA,
            pltpu.SemaphoreType.DMA,
        ],
        compiler_params=_sc_compiler_params(),
    )
    return k(vals_all, rank_flat)


_CR = 32


def _expand_body(sel_ref, tbl_ref, out_ref):
    iota40 = lax.broadcasted_iota(jnp.int32, (_TBL_ROWS, _COLS), 0)
    tbl = tbl_ref[...]
    for r in range(_CR):
        st = None
        for i in range(4):
            sel = (sel_ref[i, r, :] + _OFFS[i]).reshape(1, _COLS)
            eq = jnp.broadcast_to(sel, (_TBL_ROWS, _COLS)) == iota40
            st = eq.astype(jnp.int32) if st is None else st + eq
        tile = lax.dot_general(st.astype(jnp.float32), tbl,
                               (((0,), (0,)), ((), ())),
                               preferred_element_type=jnp.float32)
        out_ref[pl.ds(r * _COLS, _COLS), :] = tile


_HSTEPS = _HROWS // _CR


def _expand_half0(sel, table40):
    return pl.pallas_call(
        _expand_body,
        grid=(_HSTEPS,),
        in_specs=[
            pl.BlockSpec((4, _CR, _COLS), lambda j: (0, j, 0)),
            pl.BlockSpec((_TBL_ROWS, _D), lambda j: (0, 0)),
        ],
        out_specs=pl.BlockSpec((_CR * _COLS, _D), lambda j: (j, 0)),
        out_shape=jax.ShapeDtypeStruct((_N, _D), jnp.float32),
        compiler_params=pltpu.CompilerParams(
            dimension_semantics=("arbitrary",),
        ),
    )(sel, table40)


def _expand_half1_body(sel_ref, tbl_ref, carry_ref, out_ref):
    del carry_ref
    _expand_body(sel_ref, tbl_ref, out_ref)


def _expand_half1(sel, table40, carry):
    return pl.pallas_call(
        _expand_half1_body,
        grid=(_HSTEPS,),
        in_specs=[
            pl.BlockSpec((4, _CR, _COLS), lambda j: (0, j, 0)),
            pl.BlockSpec((_TBL_ROWS, _D), lambda j: (0, 0)),
            pl.BlockSpec(memory_space=pltpu.MemorySpace.HBM),
        ],
        out_specs=pl.BlockSpec((_CR * _COLS, _D), lambda j: (j + _HSTEPS, 0)),
        out_shape=jax.ShapeDtypeStruct((_N, _D), jnp.float32),
        input_output_aliases={2: 0},
        compiler_params=pltpu.CompilerParams(
            dimension_semantics=("arbitrary",),
        ),
    )(sel, table40, carry)


def kernel(block_type_grid, mask_0, mask_1, mask_2, mask_3,
           values_0, values_1, values_2, values_3,
           table_0, table_1, table_2, table_3):
    masks = jnp.stack([m.reshape(_ROWS, _COLS)
                       for m in (mask_0, mask_1, mask_2, mask_3)])
    masks_i32 = masks.astype(jnp.int32)

    vals = []
    for v, off in zip((values_0, values_1, values_2, values_3), _OFFS):
        vals.append(jnp.pad(v, (0, _VPAD - v.shape[0]),
                            constant_values=_SENT_ROW - off))
    vals_all = jnp.concatenate(vals)

    table40 = jnp.concatenate(
        [table_0, table_1, table_2, table_3,
         jnp.zeros((_TBL_ROWS - sum(_NUM_EMB), _D), jnp.float32)])

    rank_a, rank_b = _ranks(masks_i32)
    sel_a = _sc_gather(vals_all, rank_a.reshape(4 * _NH))
    sel_b = _sc_gather(vals_all, rank_b.reshape(4 * _NH))
    out_a = _expand_half0(sel_a.reshape(4, _HROWS, _COLS), table40)
    out = _expand_half1(sel_b.reshape(4, _HROWS, _COLS), table40, out_a)
    return out.reshape(_B, _W, _H, _L, _D)


# speedup vs baseline: 1.2318x; 1.2318x over previous
import dataclasses
import functools

import jax
import jax.numpy as jnp
from jax import lax
from jax.experimental import pallas as pl
from jax.experimental.pallas import tpu as pltpu
from jax.experimental.pallas import tpu_sc as plsc

_B, _W, _H, _L = 4, 32, 32, 32
_N = _B * _W * _H * _L
_D = 256
_NUM_EMB = (16, 8, 4, 6)
_OFFS = (0, 16, 24, 28)
_TBL_ROWS = 40
_SENT_ROW = 34

_COUNT = 13107
_VPAD = 13312
_SENT_SLOT = 13200

_ROWS, _COLS = 512, 256

_NC, _NS, _LANES = 2, 16, 16
_NW = _NC * _NS
_TPH = _NW // 4
_TROWS = _ROWS // _TPH


def _rank_body(m_ref, v0_ref, v1_ref, v2_ref, v3_ref, rank_ref, vp_ref):
    iota_r = lax.broadcasted_iota(jnp.int32, (_COLS, _COLS), 0)
    iota_c = lax.broadcasted_iota(jnp.int32, (_COLS, _COLS), 1)
    upper = (iota_r <= iota_c).astype(jnp.float32)
    iota_r2 = lax.broadcasted_iota(jnp.int32, (_ROWS, _ROWS), 0)
    iota_c2 = lax.broadcasted_iota(jnp.int32, (_ROWS, _ROWS), 1)
    strict_lower = (iota_c2 < iota_r2).astype(jnp.float32)

    masks = [m_ref[i] for i in range(4)]
    m_all = jnp.concatenate([m.astype(jnp.float32) for m in masks], axis=0)
    row_incl_all = lax.dot(m_all, upper, precision=lax.Precision.HIGHEST)
    tot_cols = jnp.concatenate(
        [row_incl_all[i * _ROWS:(i + 1) * _ROWS, _COLS - 1:_COLS]
         for i in range(4)], axis=1)
    col_excl_all = lax.dot(strict_lower, tot_cols,
                           precision=lax.Precision.HIGHEST)
    for i, (m, v_ref) in enumerate(zip(masks, (v0_ref, v1_ref, v2_ref,
                                               v3_ref))):
        row_incl = row_incl_all[i * _ROWS:(i + 1) * _ROWS]
        excl = row_incl - m.astype(jnp.float32) + col_excl_all[:, i:i + 1]
        rank = excl.astype(jnp.int32)
        rank_ref[i] = jnp.where(m, rank, _SENT_SLOT)
        vp_ref[i] = jnp.concatenate(
            [v_ref[...],
             jnp.full((_VPAD - _COUNT,), _SENT_ROW - _OFFS[i], jnp.int32)])


def _ranks(masks_b, values):
    return pl.pallas_call(
        _rank_body,
        out_shape=(jax.ShapeDtypeStruct((4, _ROWS, _COLS), jnp.int32),
                   jax.ShapeDtypeStruct((4, _VPAD), jnp.int32)),
    )(masks_b, *values)


def _sc_compiler_params():
    cp = pltpu.CompilerParams()
    if "needs_layout_passes" in pltpu.CompilerParams.__dataclass_fields__:
        cp = dataclasses.replace(cp, needs_layout_passes=False)
    return cp


def _gather_body(vals_hbm, idx_hbm, out_hbm, vals_v, idx_v, out_v, sem, sem2):
    wid = lax.axis_index("s") * _NC + lax.axis_index("c")
    head = wid // _TPH
    row0 = (wid % _TPH) * _TROWS
    c1 = pltpu.async_copy(vals_hbm.at[head], vals_v, sem)
    c2 = pltpu.async_copy(idx_hbm.at[head, pl.ds(row0, _TROWS)], idx_v, sem2)
    c1.wait()
    c2.wait()

    @plsc.parallel_loop(0, _TROWS, unroll=2)
    def _(r):
        for c in range(_COLS // _LANES):
            sl = pl.ds(c * _LANES, _LANES)
            out_v[r, sl] = plsc.load_gather(vals_v, [idx_v[r, sl]])

    pltpu.async_copy(out_v, out_hbm.at[head, pl.ds(row0, _TROWS)], sem).wait()


def _sc_gather(vals_pad, rank):
    mesh = plsc.VectorSubcoreMesh(core_axis_name="c", subcore_axis_name="s")
    k = pl.kernel(
        _gather_body,
        out_type=jax.ShapeDtypeStruct((4, _ROWS, _COLS), jnp.int32),
        mesh=mesh,
        scratch_types=[
            pltpu.VMEM((_VPAD,), jnp.int32),
            pltpu.VMEM((_TROWS, _COLS), jnp.int32),
            pltpu.VMEM((_TROWS, _COLS), jnp.int32),
            pltpu.SemaphoreType.DMA,
            pltpu.SemaphoreType.DMA,
        ],
        compiler_params=_sc_compiler_params(),
    )
    return k(vals_pad, rank)


_CR = 32


def _expand_body(sel_ref, tbl_ref, out_ref):
    iota40 = lax.broadcasted_iota(jnp.int32, (_TBL_ROWS, _COLS), 0)
    tbl = tbl_ref[...]
    for r in range(_CR):
        st = None
        for i in range(4):
            sel = (sel_ref[i, r, :] + _OFFS[i]).reshape(1, _COLS)
            eq = jnp.broadcast_to(sel, (_TBL_ROWS, _COLS)) == iota40
            st = eq.astype(jnp.int32) if st is None else st + eq
        tile = lax.dot_general(st.astype(jnp.float32), tbl,
                               (((0,), (0,)), ((), ())),
                               preferred_element_type=jnp.float32)
        out_ref[pl.ds(r * _COLS, _COLS), :] = tile


def _expand(sel, table40):
    return pl.pallas_call(
        _expand_body,
        grid=(_ROWS // _CR,),
        in_specs=[
            pl.BlockSpec((4, _CR, _COLS), lambda j: (0, j, 0)),
            pl.BlockSpec((_TBL_ROWS, _D), lambda j: (0, 0)),
        ],
        out_specs=pl.BlockSpec((_CR * _COLS, _D), lambda j: (j, 0)),
        out_shape=jax.ShapeDtypeStruct((_N, _D), jnp.float32),
        compiler_params=pltpu.CompilerParams(
            dimension_semantics=("arbitrary",),
        ),
    )(sel, table40)


def kernel(block_type_grid, mask_0, mask_1, mask_2, mask_3,
           values_0, values_1, values_2, values_3,
           table_0, table_1, table_2, table_3):
    table40 = jnp.concatenate(
        [table_0, table_1, table_2, table_3,
         jnp.zeros((_TBL_ROWS - sum(_NUM_EMB), _D), jnp.float32)])

    masks_b = jnp.stack([mask_0, mask_1, mask_2, mask_3]).reshape(
        4, _ROWS, _COLS)
    rank, vals_pad = _ranks(masks_b,
                            (values_0, values_1, values_2, values_3))
    sel = _sc_gather(vals_pad, rank)
    out = _expand(sel, table40)
    return out.reshape(_B, _W, _H, _L, _D)
